# Initial kernel scaffold; baseline (speedup 1.0000x reference)
import jax
import jax.numpy as jnp
from jax.experimental import pallas as pl


def kernel(xk, xv, key_past, value_past, layer_past_len):
    raise NotImplementedError("write your pallas kernel here")


# speedup vs baseline: 1.0407x; 1.0407x over previous
import jax
import jax.numpy as jnp
from jax.experimental import pallas as pl
from jax.experimental.pallas import tpu as pltpu

_B, _H, _SEQ, _D = 8, 8, 4096, 128
_S = 16
_CHUNK = 512


def _body(off_ref, xk_ref, xv_ref, kin_ref, vin_ref, kout_ref, vout_ref):
    h = pl.program_id(0)
    j = pl.program_id(1)
    kout_ref[...] = kin_ref[...]
    vout_ref[...] = vin_ref[...]
    off = off_ref[0]

    @pl.when(j == off // _CHUNK)
    def _update():
        local = off - j * _CHUNK
        for s in range(_S):
            kout_ref[:, 0, pl.ds(local + s, 1), :] = (
                xk_ref[s, h, :, :].reshape(_B, 1, _D))
            vout_ref[:, 0, pl.ds(local + s, 1), :] = (
                xv_ref[s, h, :, :].reshape(_B, 1, _D))


def kernel(xk, xv, key_past, value_past, layer_past_len):
    off = jnp.asarray(layer_past_len, jnp.int32).reshape((1,))
    out_sd = jax.ShapeDtypeStruct((_B, _H, _SEQ, _D), key_past.dtype)
    grid = (_H, _SEQ // _CHUNK)
    cache_spec = pl.BlockSpec(
        (_B, 1, _CHUNK, _D), lambda h, j: (0, h, j, 0))
    x_spec = pl.BlockSpec((_S, _H, _B, _D), lambda h, j: (0, 0, 0, 0))
    new_key, new_value = pl.pallas_call(
        _body,
        grid=grid,
        in_specs=[
            pl.BlockSpec(memory_space=pltpu.SMEM),
            x_spec,
            x_spec,
            cache_spec,
            cache_spec,
        ],
        out_specs=[cache_spec, cache_spec],
        out_shape=[out_sd, out_sd],
    )(off, xk, xv, key_past, value_past)
    return (new_key, new_value)
